# baseline (speedup 1.0000x reference)
import functools

import jax
import jax.numpy as jnp
from jax import lax
from jax.experimental import pallas as pl
from jax.experimental.pallas import tpu as pltpu
from jax.experimental.pallas import tpu_sc as plsc

_BIG = 3.0e38

_NC, _NS, _LANES = 2, 16, 16
_NW = _NC * _NS


def _prep_geo(x, scale=1.0):
    return jnp.pad(x.transpose(0, 2, 1) * scale, ((0, 0), (0, 5), (0, 0)))


def _prep_q(x):
    xt = x.transpose(0, 2, 1)
    n2 = jnp.sum(x * x, axis=-1)[:, None, :]
    return jnp.concatenate(
        [xt, n2, jnp.ones_like(n2), jnp.zeros_like(xt)], axis=1)


def _prep_k(x):
    xt = -2.0 * x.transpose(0, 2, 1)
    n2 = jnp.sum(x * x, axis=-1)[:, None, :]
    return jnp.concatenate(
        [xt, jnp.ones_like(n2), n2, jnp.zeros_like(xt)], axis=1)


def _stage_body(qx_ref, qn_ref, kx_ref, kn_ref, pq_ref, pk_ref, out_ref, *, NT, S):
    ax = qx_ref[0]
    an = qn_ref[0]
    bx = kx_ref[0]
    bn = kn_ref[0]

    dnums = (((0,), (0,)), ((), ()))

    m2dotx = lax.dot_general(ax, bx, dnums, preferred_element_type=jnp.float32)
    na = jnp.sum(ax * ax, axis=0)[:, None]
    nb = 0.25 * jnp.sum(bx * bx, axis=0)[None, :]
    dx = jnp.sqrt(jnp.clip(na + (nb + m2dotx), 1e-12))

    m2dotn = lax.dot_general(an, bn, dnums, preferred_element_type=jnp.float32)
    nna = jnp.sum(an * an, axis=0)[:, None]
    nnb = 0.25 * jnp.sum(bn * bn, axis=0)[None, :]
    dn = jnp.sqrt(jnp.clip(nna + (nnb + m2dotn), 1e-12))

    dist = dx + jax.nn.sigmoid(dn)

    work = dist
    mins = []
    masks = []
    for k in range(3):
        m = jnp.min(work, axis=1, keepdims=True)
        mask = work == m
        mins.append(m)
        masks.append(mask)
        if k < 2:
            work = jnp.where(mask, _BIG, work)

    recips = [1.0 / (m + 1e-8) for m in mins]
    norm = recips[0] + recips[1] + recips[2]
    sel = (jnp.where(masks[0], recips[0] / norm, 0.0)
           + jnp.where(masks[1], recips[1] / norm, 0.0)
           + jnp.where(masks[2], recips[2] / norm, 0.0))

    interp = lax.dot_general(sel, pk_ref[0], (((1,), (0,)), ((), ())),
                             preferred_element_type=jnp.float32)
    p1 = pq_ref[0]
    out_ref[0] = (jnp.maximum(p1, interp) + (p1 + interp) * 0.5) * 0.5


def _stage_tc(qx, qn, kx, kn, pq, pk, NT, interpret=False):
    B, _, N = qx.shape
    S = kx.shape[2]
    D = pq.shape[2]
    grid = (B, N // NT)
    body = functools.partial(_stage_body, NT=NT, S=S)
    return pl.pallas_call(
        body,
        grid=grid,
        in_specs=[
            pl.BlockSpec((1, 8, NT), lambda b, n: (b, 0, n)),
            pl.BlockSpec((1, 8, NT), lambda b, n: (b, 0, n)),
            pl.BlockSpec((1, 8, S), lambda b, n: (b, 0, 0)),
            pl.BlockSpec((1, 8, S), lambda b, n: (b, 0, 0)),
            pl.BlockSpec((1, NT, D), lambda b, n: (b, n, 0)),
            pl.BlockSpec((1, S, D), lambda b, n: (b, 0, 0)),
        ],
        out_specs=pl.BlockSpec((1, NT, D), lambda b, n: (b, n, 0)),
        out_shape=jax.ShapeDtypeStruct((B, N, D), jnp.float32),
        interpret=interpret,
    )(qx, qn, kx, kn, pq, pk)


def _stage_idx_body(qx_ref, qn_ref, kx_ref, kn_ref, idx_ref, wts_ref, *, NT, S):
    ax = qx_ref[0]
    an = qn_ref[0]
    bx = kx_ref[0]
    bn = kn_ref[0]

    dnums = (((0,), (0,)), ((), ()))

    m2dotx = lax.dot_general(ax, bx, dnums, preferred_element_type=jnp.float32)
    na = jnp.sum(ax * ax, axis=0)[:, None]
    nb = 0.25 * jnp.sum(bx * bx, axis=0)[None, :]
    dx = jnp.sqrt(jnp.clip(na + (nb + m2dotx), 1e-12))

    m2dotn = lax.dot_general(an, bn, dnums, preferred_element_type=jnp.float32)
    nna = jnp.sum(an * an, axis=0)[:, None]
    nnb = 0.25 * jnp.sum(bn * bn, axis=0)[None, :]
    dn = jnp.sqrt(jnp.clip(nna + (nnb + m2dotn), 1e-12))

    dist = dx + jax.nn.sigmoid(dn)

    iota = lax.broadcasted_iota(jnp.int32, (NT, S), 1)
    b = pl.program_id(0)
    work = dist
    mins = []
    idxs = []
    for k in range(3):
        m = jnp.min(work, axis=1, keepdims=True)
        mask = work == m
        im = jnp.min(jnp.where(mask, iota, S), axis=1, keepdims=True)
        mins.append(m)
        idxs.append(im + b * S)
        if k < 2:
            work = jnp.where(mask, _BIG, work)

    recips = [1.0 / (m + 1e-8) for m in mins]
    norm = recips[0] + recips[1] + recips[2]
    ws = [r / norm for r in recips]

    idx_ref[0] = jnp.concatenate(idxs + idxs + [idxs[0], idxs[1]], axis=1)
    wts_ref[0] = jnp.concatenate(
        [jnp.broadcast_to(w, (NT, 16)) for w in ws], axis=1)


def _stage_tc_idx(qx, qn, kx, kn, NT):
    B, _, N = qx.shape
    S = kx.shape[2]
    grid = (B, N // NT)
    body = functools.partial(_stage_idx_body, NT=NT, S=S)
    return pl.pallas_call(
        body,
        grid=grid,
        in_specs=[
            pl.BlockSpec((1, 8, NT), lambda b, n: (b, 0, n)),
            pl.BlockSpec((1, 8, NT), lambda b, n: (b, 0, n)),
            pl.BlockSpec((1, 8, S), lambda b, n: (b, 0, 0)),
            pl.BlockSpec((1, 8, S), lambda b, n: (b, 0, 0)),
        ],
        out_specs=[
            pl.BlockSpec((1, NT, 8), lambda b, n: (b, n, 0)),
            pl.BlockSpec((1, NT, 48), lambda b, n: (b, n, 0)),
        ],
        out_shape=[
            jax.ShapeDtypeStruct((B, N, 8), jnp.int32),
            jax.ShapeDtypeStruct((B, N, 48), jnp.float32),
        ],
    )(qx, qn, kx, kn)


def _sc_gather_combine(gidx, wts, table, pq):
    BN, D = pq.shape
    Q = BN // _NW
    C = min(128, Q)
    chunks = Q // C
    mesh = plsc.VectorSubcoreMesh(core_axis_name="c", subcore_axis_name="s",
                                  num_cores=_NC, num_subcores=_NS)

    @functools.partial(
        pl.kernel,
        out_type=jax.ShapeDtypeStruct((BN, D), jnp.float32),
        mesh=mesh,
        scratch_types=[
            pltpu.VMEM((2, 3 * C), jnp.int32),
            pltpu.VMEM((2, 3 * C, _LANES), jnp.float32),
            pltpu.VMEM((2, 3 * C, D), jnp.float32),
            pltpu.VMEM((2, C, D), jnp.float32),
            pltpu.VMEM((C, D), jnp.float32),
            pltpu.SemaphoreType.DMA,
            pltpu.SemaphoreType.DMA,
            pltpu.SemaphoreType.DMA,
            pltpu.SemaphoreType.DMA,
        ],
        compiler_params=pltpu.CompilerParams(use_tc_tiling_on_sc=False),
    )
    def sc_kernel(gidx_hbm, wts_hbm, table_hbm, pq_hbm, out_hbm,
                  idx_v, wts_v, rows_v, p1_v, out_v,
                  sem_in0, sem_in1, sem_g0, sem_g1):
        wid = lax.axis_index("s") * _NC + lax.axis_index("c")
        sem_in = [sem_in0, sem_in1]
        sem_g = [sem_g0, sem_g1]

        def start_inputs(c, buf):
            base = wid * Q + c * C
            return [
                pltpu.async_copy(gidx_hbm.at[pl.ds(base * 3, 3 * C)],
                                 idx_v.at[buf], sem_in[buf]),
                pltpu.async_copy(wts_hbm.at[pl.ds(base * 3, 3 * C)],
                                 wts_v.at[buf], sem_in[buf]),
                pltpu.async_copy(pq_hbm.at[pl.ds(base, C)],
                                 p1_v.at[buf], sem_in[buf]),
            ]

        def start_gathers(buf):
            return [pltpu.async_copy(table_hbm.at[idx_v.at[buf, pl.ds(k * C, C)]],
                                     rows_v.at[buf, pl.ds(k * C, C)],
                                     sem_g[buf])
                    for k in range(3)]

        for cp in start_inputs(0, 0):
            cp.wait()
        for _cp in start_gathers(0):
            pass

        def process(c, buf, nxt):
            base = wid * Q + c * C

            @pl.when(c + 1 < chunks)
            def _():
                for cp in start_inputs(c + 1, nxt):
                    cp.wait()
                for _g in start_gathers(nxt):
                    pass

            for k in range(3):
                pltpu.make_async_copy(table_hbm.at[idx_v.at[buf, pl.ds(k * C, C)]],
                                      rows_v.at[buf, pl.ds(k * C, C)],
                                      sem_g[buf]).wait()

            def one_q(q):
                r = 3 * q
                w0 = wts_v[buf, r, :]
                w1 = wts_v[buf, r + 1, :]
                w2 = wts_v[buf, r + 2, :]
                for dc in range(D // _LANES):
                    sl = pl.ds(dc * _LANES, _LANES)
                    acc = (w0 * rows_v[buf, r, sl]
                           + w1 * rows_v[buf, r + 1, sl]
                           + w2 * rows_v[buf, r + 2, sl])
                    p1v = p1_v[buf, q, sl]
                    out_v[q, sl] = (jnp.maximum(p1v, acc)
                                    + (p1v + acc) * 0.5) * 0.5

            def q_body(i, carry2):
                one_q(2 * i)
                one_q(2 * i + 1)
                return carry2

            lax.fori_loop(0, C // 2, q_body, 0)
            pltpu.sync_copy(out_v, out_hbm.at[pl.ds(base, C)])

        def pair_body(i, carry):
            process(2 * i, 0, 1)
            process(2 * i + 1, 1, 0)
            return carry

        lax.fori_loop(0, chunks // 2, pair_body, 0)

    return sc_kernel(gidx, wts, table, pq)


def _propagate(xyz_q, xyz_k, n_q, n_k, feats_q, feats_k, NT):
    B, N, D = feats_q.shape
    S = xyz_k.shape[1]
    idx, wts = _stage_tc_idx(_prep_geo(xyz_q), _prep_geo(n_q),
                             _prep_geo(xyz_k, -2.0), _prep_geo(n_k, -2.0), NT)
    gidx = idx[..., :3].reshape(3 * B * N)
    wtsf = wts.reshape(3 * B * N, 16)
    out = _sc_gather_combine(gidx, wtsf, feats_k.reshape(B * S, D),
                             feats_q.reshape(B * N, D))
    return out.reshape(B, N, D)


def kernel(xyz0, xyz1, xyz2, normal0, normal1, normal2, points0, points1, points2):
    x = _propagate(xyz1, xyz2, normal1, normal2, points1, points2, NT=256)
    x = _propagate(xyz0, xyz1, normal0, normal1, points0, x, NT=256)
    return x


# speedup vs baseline: 1.0691x; 1.0691x over previous
import functools

import jax
import jax.numpy as jnp
from jax import lax
from jax.experimental import pallas as pl
from jax.experimental.pallas import tpu as pltpu
from jax.experimental.pallas import tpu_sc as plsc

_BIG = 3.0e38

_NC, _NS, _LANES = 2, 16, 16
_NW = _NC * _NS


def _prep_geo(x, scale=1.0):
    return jnp.pad(x.transpose(0, 2, 1) * scale, ((0, 0), (0, 5), (0, 0)))


def _stage_body(qx_ref, qn_ref, kx_ref, kn_ref, pq_ref, pk_ref, out_ref, *, NT, S):
    ax = qx_ref[0]
    an = qn_ref[0]
    bx = kx_ref[0]
    bn = kn_ref[0]

    dnums = (((0,), (0,)), ((), ()))

    m2dotx = lax.dot_general(ax, bx, dnums, preferred_element_type=jnp.float32)
    na = jnp.sum(ax * ax, axis=0)[:, None]
    nb = 0.25 * jnp.sum(bx * bx, axis=0)[None, :]
    dx = jnp.sqrt(jnp.clip(na + (nb + m2dotx), 1e-12))

    m2dotn = lax.dot_general(an, bn, dnums, preferred_element_type=jnp.float32)
    nna = jnp.sum(an * an, axis=0)[:, None]
    nnb = 0.25 * jnp.sum(bn * bn, axis=0)[None, :]
    dn = jnp.sqrt(jnp.clip(nna + (nnb + m2dotn), 1e-12))

    dist = dx + jax.nn.sigmoid(dn)

    work = dist
    mins = []
    masks = []
    for k in range(3):
        m = jnp.min(work, axis=1, keepdims=True)
        mask = work == m
        mins.append(m)
        masks.append(mask)
        if k < 2:
            work = jnp.where(mask, _BIG, work)

    recips = [1.0 / (m + 1e-8) for m in mins]
    norm = recips[0] + recips[1] + recips[2]
    sel = (jnp.where(masks[0], recips[0] / norm, 0.0)
           + jnp.where(masks[1], recips[1] / norm, 0.0)
           + jnp.where(masks[2], recips[2] / norm, 0.0))

    interp = lax.dot_general(sel, pk_ref[0], (((1,), (0,)), ((), ())),
                             preferred_element_type=jnp.float32)
    p1 = pq_ref[0]
    out_ref[0] = (jnp.maximum(p1, interp) + (p1 + interp) * 0.5) * 0.5


def _stage_tc(qx, qn, kx, kn, pq, pk, NT, interpret=False):
    B, _, N = qx.shape
    S = kx.shape[2]
    D = pq.shape[2]
    grid = (B, N // NT)
    body = functools.partial(_stage_body, NT=NT, S=S)
    return pl.pallas_call(
        body,
        grid=grid,
        in_specs=[
            pl.BlockSpec((1, 8, NT), lambda b, n: (b, 0, n)),
            pl.BlockSpec((1, 8, NT), lambda b, n: (b, 0, n)),
            pl.BlockSpec((1, 8, S), lambda b, n: (b, 0, 0)),
            pl.BlockSpec((1, 8, S), lambda b, n: (b, 0, 0)),
            pl.BlockSpec((1, NT, D), lambda b, n: (b, n, 0)),
            pl.BlockSpec((1, S, D), lambda b, n: (b, 0, 0)),
        ],
        out_specs=pl.BlockSpec((1, NT, D), lambda b, n: (b, n, 0)),
        out_shape=jax.ShapeDtypeStruct((B, N, D), jnp.float32),
        interpret=interpret,
    )(qx, qn, kx, kn, pq, pk)


def _stage_idx_body(qx_ref, qn_ref, kx_ref, kn_ref, idx_ref, wts_ref, *, NT, S):
    ax = qx_ref[0]
    an = qn_ref[0]
    bx = kx_ref[0]
    bn = kn_ref[0]

    dnums = (((0,), (0,)), ((), ()))

    m2dotx = lax.dot_general(ax, bx, dnums, preferred_element_type=jnp.float32)
    na = jnp.sum(ax * ax, axis=0)[:, None]
    nb = 0.25 * jnp.sum(bx * bx, axis=0)[None, :]
    dx = jnp.sqrt(jnp.clip(na + (nb + m2dotx), 1e-12))

    m2dotn = lax.dot_general(an, bn, dnums, preferred_element_type=jnp.float32)
    nna = jnp.sum(an * an, axis=0)[:, None]
    nnb = 0.25 * jnp.sum(bn * bn, axis=0)[None, :]
    dn = jnp.sqrt(jnp.clip(nna + (nnb + m2dotn), 1e-12))

    dist = dx + jax.nn.sigmoid(dn)

    iota_f = lax.broadcasted_iota(jnp.int32, (NT, S), 1).astype(jnp.float32)
    b = pl.program_id(0)
    work = dist
    mins = []
    idxs = []
    for k in range(3):
        m = jnp.min(work, axis=1, keepdims=True)
        mask = work == m
        imf = jnp.min(jnp.where(mask, iota_f, _BIG), axis=1, keepdims=True)
        mins.append(m)
        idxs.append(imf.astype(jnp.int32) + b * S)
        if k < 2:
            work = jnp.where(mask, _BIG, work)

    recips = [1.0 / (m + 1e-8) for m in mins]
    norm = recips[0] + recips[1] + recips[2]
    ws = [r / norm for r in recips]

    idx_ref[0] = jnp.concatenate(idxs + idxs + [idxs[0], idxs[1]], axis=1)
    wts_ref[0] = jnp.concatenate(
        [jnp.broadcast_to(w, (NT, 16)) for w in ws], axis=1)


def _stage_tc_idx(qx, qn, kx, kn, NT):
    B, _, N = qx.shape
    S = kx.shape[2]
    grid = (B, N // NT)
    body = functools.partial(_stage_idx_body, NT=NT, S=S)
    return pl.pallas_call(
        body,
        grid=grid,
        in_specs=[
            pl.BlockSpec((1, 8, NT), lambda b, n: (b, 0, n)),
            pl.BlockSpec((1, 8, NT), lambda b, n: (b, 0, n)),
            pl.BlockSpec((1, 8, S), lambda b, n: (b, 0, 0)),
            pl.BlockSpec((1, 8, S), lambda b, n: (b, 0, 0)),
        ],
        out_specs=[
            pl.BlockSpec((1, NT, 8), lambda b, n: (b, n, 0)),
            pl.BlockSpec((1, NT, 48), lambda b, n: (b, n, 0)),
        ],
        out_shape=[
            jax.ShapeDtypeStruct((B, N, 8), jnp.int32),
            jax.ShapeDtypeStruct((B, N, 48), jnp.float32),
        ],
    )(qx, qn, kx, kn)


def _sc_gather_combine(gidx, wts, table, pq):
    BN, D = pq.shape
    Q = BN // _NW
    C = min(128, Q)
    chunks = Q // C
    mesh = plsc.VectorSubcoreMesh(core_axis_name="c", subcore_axis_name="s",
                                  num_cores=_NC, num_subcores=_NS)

    @functools.partial(
        pl.kernel,
        out_type=jax.ShapeDtypeStruct((BN, D), jnp.float32),
        mesh=mesh,
        scratch_types=[
            pltpu.VMEM((2, 3 * C), jnp.int32),
            pltpu.VMEM((2, 3 * C, _LANES), jnp.float32),
            pltpu.VMEM((2, 3 * C, D), jnp.float32),
            pltpu.VMEM((2, C, D), jnp.float32),
            pltpu.VMEM((C, D), jnp.float32),
            pltpu.SemaphoreType.DMA,
            pltpu.SemaphoreType.DMA,
            pltpu.SemaphoreType.DMA,
            pltpu.SemaphoreType.DMA,
        ],
        compiler_params=pltpu.CompilerParams(use_tc_tiling_on_sc=False),
    )
    def sc_kernel(gidx_hbm, wts_hbm, table_hbm, pq_hbm, out_hbm,
                  idx_v, wts_v, rows_v, p1_v, out_v,
                  sem_in0, sem_in1, sem_g0, sem_g1):
        wid = lax.axis_index("s") * _NC + lax.axis_index("c")
        sem_in = [sem_in0, sem_in1]
        sem_g = [sem_g0, sem_g1]

        def start_inputs(c, buf):
            base = wid * Q + c * C
            return [
                pltpu.async_copy(gidx_hbm.at[pl.ds(base * 3, 3 * C)],
                                 idx_v.at[buf], sem_in[buf]),
                pltpu.async_copy(wts_hbm.at[pl.ds(base * 3, 3 * C)],
                                 wts_v.at[buf], sem_in[buf]),
                pltpu.async_copy(pq_hbm.at[pl.ds(base, C)],
                                 p1_v.at[buf], sem_in[buf]),
            ]

        def start_gathers(buf):
            return [pltpu.async_copy(table_hbm.at[idx_v.at[buf, pl.ds(k * C, C)]],
                                     rows_v.at[buf, pl.ds(k * C, C)],
                                     sem_g[buf])
                    for k in range(3)]

        for cp in start_inputs(0, 0):
            cp.wait()
        for _cp in start_gathers(0):
            pass

        def process(c, buf, nxt):
            base = wid * Q + c * C

            @pl.when(c + 1 < chunks)
            def _():
                for cp in start_inputs(c + 1, nxt):
                    cp.wait()
                for _g in start_gathers(nxt):
                    pass

            for k in range(3):
                pltpu.make_async_copy(table_hbm.at[idx_v.at[buf, pl.ds(k * C, C)]],
                                      rows_v.at[buf, pl.ds(k * C, C)],
                                      sem_g[buf]).wait()

            def one_q(q):
                r = 3 * q
                w0 = wts_v[buf, r, :]
                w1 = wts_v[buf, r + 1, :]
                w2 = wts_v[buf, r + 2, :]
                for dc in range(D // _LANES):
                    sl = pl.ds(dc * _LANES, _LANES)
                    acc = (w0 * rows_v[buf, r, sl]
                           + w1 * rows_v[buf, r + 1, sl]
                           + w2 * rows_v[buf, r + 2, sl])
                    p1v = p1_v[buf, q, sl]
                    out_v[q, sl] = (jnp.maximum(p1v, acc)
                                    + (p1v + acc) * 0.5) * 0.5

            def q_body(i, carry2):
                one_q(2 * i)
                one_q(2 * i + 1)
                return carry2

            lax.fori_loop(0, C // 2, q_body, 0)
            pltpu.sync_copy(out_v, out_hbm.at[pl.ds(base, C)])

        def pair_body(i, carry):
            process(2 * i, 0, 1)
            process(2 * i + 1, 1, 0)
            return carry

        lax.fori_loop(0, chunks // 2, pair_body, 0)

    return sc_kernel(gidx, wts, table, pq)


def _propagate(xyz_q, xyz_k, n_q, n_k, feats_q, feats_k, NT):
    B, N, D = feats_q.shape
    S = xyz_k.shape[1]
    idx, wts = _stage_tc_idx(_prep_geo(xyz_q), _prep_geo(n_q),
                             _prep_geo(xyz_k, -2.0), _prep_geo(n_k, -2.0), NT)
    gidx = idx[..., :3].reshape(3 * B * N)
    wtsf = wts.reshape(3 * B * N, 16)
    out = _sc_gather_combine(gidx, wtsf, feats_k.reshape(B * S, D),
                             feats_q.reshape(B * N, D))
    return out.reshape(B, N, D)


def kernel(xyz0, xyz1, xyz2, normal0, normal1, normal2, points0, points1, points2):
    x = _propagate(xyz1, xyz2, normal1, normal2, points1, points2, NT=256)
    x = _propagate(xyz0, xyz1, normal0, normal1, points0, x, NT=256)
    return x


# speedup vs baseline: 1.1101x; 1.0384x over previous
import functools

import jax
import jax.numpy as jnp
from jax import lax
from jax.experimental import pallas as pl
from jax.experimental.pallas import tpu as pltpu
from jax.experimental.pallas import tpu_sc as plsc

_BIG = 3.0e38

_NC, _NS, _LANES = 2, 16, 16
_NW = _NC * _NS


def _prep_geo(x, scale=1.0):
    return jnp.pad(x.transpose(0, 2, 1) * scale, ((0, 0), (0, 5), (0, 0)))


def _stage_body(qx_ref, qn_ref, kx_ref, kn_ref, pq_ref, pk_ref, out_ref, *, NT, S):
    ax = qx_ref[0]
    an = qn_ref[0]
    bx = kx_ref[0]
    bn = kn_ref[0]

    dnums = (((0,), (0,)), ((), ()))

    m2dotx = lax.dot_general(ax, bx, dnums, preferred_element_type=jnp.float32)
    na = jnp.sum(ax * ax, axis=0)[:, None]
    nb = 0.25 * jnp.sum(bx * bx, axis=0)[None, :]
    dx = jnp.sqrt(jnp.clip(na + (nb + m2dotx), 1e-12))

    m2dotn = lax.dot_general(an, bn, dnums, preferred_element_type=jnp.float32)
    nna = jnp.sum(an * an, axis=0)[:, None]
    nnb = 0.25 * jnp.sum(bn * bn, axis=0)[None, :]
    dn = jnp.sqrt(jnp.clip(nna + (nnb + m2dotn), 1e-12))

    dist = dx + jax.nn.sigmoid(dn)

    work = dist
    mins = []
    masks = []
    for k in range(3):
        m = jnp.min(work, axis=1, keepdims=True)
        mask = work == m
        mins.append(m)
        masks.append(mask)
        if k < 2:
            work = jnp.where(mask, _BIG, work)

    recips = [1.0 / (m + 1e-8) for m in mins]
    norm = recips[0] + recips[1] + recips[2]
    sel = (jnp.where(masks[0], recips[0] / norm, 0.0)
           + jnp.where(masks[1], recips[1] / norm, 0.0)
           + jnp.where(masks[2], recips[2] / norm, 0.0))

    interp = lax.dot_general(sel, pk_ref[0], (((1,), (0,)), ((), ())),
                             preferred_element_type=jnp.float32)
    p1 = pq_ref[0]
    out_ref[0] = (jnp.maximum(p1, interp) + (p1 + interp) * 0.5) * 0.5


def _stage_tc(qx, qn, kx, kn, pq, pk, NT, interpret=False):
    B, _, N = qx.shape
    S = kx.shape[2]
    D = pq.shape[2]
    grid = (B, N // NT)
    body = functools.partial(_stage_body, NT=NT, S=S)
    return pl.pallas_call(
        body,
        grid=grid,
        in_specs=[
            pl.BlockSpec((1, 8, NT), lambda b, n: (b, 0, n)),
            pl.BlockSpec((1, 8, NT), lambda b, n: (b, 0, n)),
            pl.BlockSpec((1, 8, S), lambda b, n: (b, 0, 0)),
            pl.BlockSpec((1, 8, S), lambda b, n: (b, 0, 0)),
            pl.BlockSpec((1, NT, D), lambda b, n: (b, n, 0)),
            pl.BlockSpec((1, S, D), lambda b, n: (b, 0, 0)),
        ],
        out_specs=pl.BlockSpec((1, NT, D), lambda b, n: (b, n, 0)),
        out_shape=jax.ShapeDtypeStruct((B, N, D), jnp.float32),
        interpret=interpret,
    )(qx, qn, kx, kn, pq, pk)


def _stage_idx_body(qx_ref, qn_ref, kx_ref, kn_ref, idx_ref, wts_ref, *, NT, S):
    ax = qx_ref[0]
    an = qn_ref[0]
    bx = kx_ref[0]
    bn = kn_ref[0]

    dnums = (((0,), (0,)), ((), ()))

    m2dotx = lax.dot_general(ax, bx, dnums, preferred_element_type=jnp.float32)
    na = jnp.sum(ax * ax, axis=0)[:, None]
    nb = 0.25 * jnp.sum(bx * bx, axis=0)[None, :]
    dx = jnp.sqrt(jnp.clip(na + (nb + m2dotx), 1e-12))

    m2dotn = lax.dot_general(an, bn, dnums, preferred_element_type=jnp.float32)
    nna = jnp.sum(an * an, axis=0)[:, None]
    nnb = 0.25 * jnp.sum(bn * bn, axis=0)[None, :]
    dn = jnp.sqrt(jnp.clip(nna + (nnb + m2dotn), 1e-12))

    dist = dx + jax.nn.sigmoid(dn)

    iota_f = lax.broadcasted_iota(jnp.int32, (NT, S), 1).astype(jnp.float32)
    b = pl.program_id(0)
    work = dist
    mins = []
    idxs = []
    for k in range(3):
        m = jnp.min(work, axis=1, keepdims=True)
        mask = work == m
        imf = jnp.min(jnp.where(mask, iota_f, _BIG), axis=1, keepdims=True)
        mins.append(m)
        idxs.append(imf.astype(jnp.int32) + b * S)
        if k < 2:
            work = jnp.where(mask, _BIG, work)

    recips = [1.0 / (m + 1e-8) for m in mins]
    norm = recips[0] + recips[1] + recips[2]
    ws = [r / norm for r in recips]

    idx_ref[0] = jnp.concatenate(idxs + idxs + [idxs[0], idxs[1]], axis=1)
    wts_ref[0] = jnp.concatenate(
        [jnp.broadcast_to(w, (NT, 16)) for w in ws], axis=1)


def _stage_tc_idx(qx, qn, kx, kn, NT):
    B, _, N = qx.shape
    S = kx.shape[2]
    grid = (B, N // NT)
    body = functools.partial(_stage_idx_body, NT=NT, S=S)
    return pl.pallas_call(
        body,
        grid=grid,
        in_specs=[
            pl.BlockSpec((1, 8, NT), lambda b, n: (b, 0, n)),
            pl.BlockSpec((1, 8, NT), lambda b, n: (b, 0, n)),
            pl.BlockSpec((1, 8, S), lambda b, n: (b, 0, 0)),
            pl.BlockSpec((1, 8, S), lambda b, n: (b, 0, 0)),
        ],
        out_specs=[
            pl.BlockSpec((1, NT, 8), lambda b, n: (b, n, 0)),
            pl.BlockSpec((1, NT, 48), lambda b, n: (b, n, 0)),
        ],
        out_shape=[
            jax.ShapeDtypeStruct((B, N, 8), jnp.int32),
            jax.ShapeDtypeStruct((B, N, 48), jnp.float32),
        ],
    )(qx, qn, kx, kn)


def _sc_gather_combine(gidx, wts, table, pq):
    BN, D = pq.shape
    Q = BN // _NW
    C = min(128, Q)
    chunks = Q // C
    mesh = plsc.VectorSubcoreMesh(core_axis_name="c", subcore_axis_name="s",
                                  num_cores=_NC, num_subcores=_NS)

    @functools.partial(
        pl.kernel,
        out_type=jax.ShapeDtypeStruct((BN, D), jnp.float32),
        mesh=mesh,
        scratch_types=[
            pltpu.VMEM((2, 3 * C), jnp.int32),
            pltpu.VMEM((2, 3 * C, _LANES), jnp.float32),
            pltpu.VMEM((2, 3 * C, D), jnp.float32),
            pltpu.VMEM((2, C, D), jnp.float32),
            pltpu.VMEM((2, C, D), jnp.float32),
            pltpu.SemaphoreType.DMA,
            pltpu.SemaphoreType.DMA,
            pltpu.SemaphoreType.DMA,
            pltpu.SemaphoreType.DMA,
            pltpu.SemaphoreType.DMA,
            pltpu.SemaphoreType.DMA,
        ],
        compiler_params=pltpu.CompilerParams(use_tc_tiling_on_sc=False),
    )
    def sc_kernel(gidx_hbm, wts_hbm, table_hbm, pq_hbm, out_hbm,
                  idx_v, wts_v, rows_v, p1_v, out_v,
                  sem_in0, sem_in1, sem_g0, sem_g1, sem_o0, sem_o1):
        wid = lax.axis_index("s") * _NC + lax.axis_index("c")
        sem_in = [sem_in0, sem_in1]
        sem_g = [sem_g0, sem_g1]
        sem_o = [sem_o0, sem_o1]

        def _input_copies(c, buf, issue):
            base = wid * Q + c * C
            mk = pltpu.async_copy if issue else pltpu.make_async_copy
            return [
                mk(gidx_hbm.at[pl.ds(base * 3, 3 * C)],
                   idx_v.at[buf], sem_in[buf]),
                mk(wts_hbm.at[pl.ds(base * 3, 3 * C)],
                   wts_v.at[buf], sem_in[buf]),
                mk(pq_hbm.at[pl.ds(base, C)],
                   p1_v.at[buf], sem_in[buf]),
            ]

        def start_inputs(c, buf):
            return _input_copies(c, buf, True)

        def wait_inputs(c, buf):
            for cp in _input_copies(c, buf, False):
                cp.wait()

        def start_gathers(buf):
            return [pltpu.async_copy(table_hbm.at[idx_v.at[buf, pl.ds(k * C, C)]],
                                     rows_v.at[buf, pl.ds(k * C, C)],
                                     sem_g[buf])
                    for k in range(3)]

        def out_issue(c, buf):
            base = wid * Q + c * C
            pltpu.async_copy(out_v.at[buf], out_hbm.at[pl.ds(base, C)],
                             sem_o[buf])

        def out_wait(c, buf):
            base = wid * Q + c * C
            pltpu.make_async_copy(out_v.at[buf], out_hbm.at[pl.ds(base, C)],
                                  sem_o[buf]).wait()

        for cp in start_inputs(0, 0):
            cp.wait()
        for _cp in start_gathers(0):
            pass
        if chunks > 1:
            for _cp in start_inputs(1, 1):
                pass

        def process(c, buf, nxt):
            @pl.when(c + 1 < chunks)
            def _():
                wait_inputs(c + 1, nxt)
                for _g in start_gathers(nxt):
                    pass

            for k in range(3):
                pltpu.make_async_copy(table_hbm.at[idx_v.at[buf, pl.ds(k * C, C)]],
                                      rows_v.at[buf, pl.ds(k * C, C)],
                                      sem_g[buf]).wait()

            def one_q(q):
                r = 3 * q
                w0 = wts_v[buf, r, :]
                w1 = wts_v[buf, r + 1, :]
                w2 = wts_v[buf, r + 2, :]
                for dc in range(D // _LANES):
                    sl = pl.ds(dc * _LANES, _LANES)
                    acc = (w0 * rows_v[buf, r, sl]
                           + w1 * rows_v[buf, r + 1, sl]
                           + w2 * rows_v[buf, r + 2, sl])
                    p1v = p1_v[buf, q, sl]
                    out_v[buf, q, sl] = (jnp.maximum(p1v, acc)
                                         + (p1v + acc) * 0.5) * 0.5

            def q_body(i, carry2):
                one_q(2 * i)
                one_q(2 * i + 1)
                return carry2

            lax.fori_loop(0, C // 2, q_body, 0)

            out_issue(c, buf)
            out_wait(c, buf)

            @pl.when(c + 2 < chunks)
            def _():
                for _cp in start_inputs(c + 2, buf):
                    pass

        def pair_body(i, carry):
            process(2 * i, 0, 1)
            process(2 * i + 1, 1, 0)
            return carry

        lax.fori_loop(0, chunks // 2, pair_body, 0)

    return sc_kernel(gidx, wts, table, pq)


def _propagate(xyz_q, xyz_k, n_q, n_k, feats_q, feats_k, NT):
    B, N, D = feats_q.shape
    S = xyz_k.shape[1]
    idx, wts = _stage_tc_idx(_prep_geo(xyz_q), _prep_geo(n_q),
                             _prep_geo(xyz_k, -2.0), _prep_geo(n_k, -2.0), NT)
    gidx = idx[..., :3].reshape(3 * B * N)
    wtsf = wts.reshape(3 * B * N, 16)
    out = _sc_gather_combine(gidx, wtsf, feats_k.reshape(B * S, D),
                             feats_q.reshape(B * N, D))
    return out.reshape(B, N, D)


def kernel(xyz0, xyz1, xyz2, normal0, normal1, normal2, points0, points1, points2):
    x = _propagate(xyz1, xyz2, normal1, normal2, points1, points2, NT=512)
    x = _propagate(xyz0, xyz1, normal0, normal1, points0, x, NT=512)
    return x
